# baseline (speedup 1.0000x reference)
import functools

import jax
import jax.numpy as jnp
from jax import lax
from jax.experimental import pallas as pl
from jax.experimental.pallas import tpu as pltpu
from jax.experimental.pallas import tpu_sc as plsc

H = 272
W = 272
C = 2
N = 500
HW = H * W
L = 16
NPAD = 512
TAIL = N - (NPAD - L)
NG = NPAD // L
DIV17_MAGIC = 61681
DIV17_SHIFT = 20

_mesh = plsc.VectorSubcoreMesh(
    core_axis_name="c", subcore_axis_name="s", num_cores=1, num_subcores=1
)


@functools.partial(
    pl.kernel,
    mesh=_mesh,
    out_type=jax.ShapeDtypeStruct((1,), jnp.float32),
    compiler_params=pltpu.CompilerParams(needs_layout_passes=False),
    scratch_types=[
        pltpu.VMEM((H, W), jnp.float32),
        pltpu.VMEM((NPAD,), jnp.int32),
        pltpu.VMEM((NPAD,), jnp.int32),
        pltpu.VMEM((NPAD,), jnp.int32),
        pltpu.VMEM((C, N), jnp.float32),
        pltpu.VMEM((NPAD,), jnp.float32),
        pltpu.VMEM((L,), jnp.float32),
        pltpu.VMEM((L,), jnp.float32),
        pltpu.SemaphoreType.DMA,
        pltpu.SemaphoreType.DMA,
    ],
)
def _sc_l1_loss(tab_hbm, t_hbm, m_hbm, idx_hbm, out_hbm,
                ch_v, idx_v, row_v, col_v, t_v, m_v, red_v, out_v, psem, isem):

    def lane_total(x):
        for shift in (8, 4, 2, 1):
            red_v[...] = x
            perm = lax.iota(jnp.int32, L) ^ shift
            x = x + plsc.load_gather(red_v, [perm])
        return x

    lane = lax.iota(jnp.int32, L)
    tail_sl = pl.ds(NPAD - L, L)
    tail_col = jnp.where(lane < TAIL, (NPAD - L) + lane, 0)

    cp0 = pltpu.async_copy(tab_hbm.at[0], ch_v, psem)
    t_cp = pltpu.async_copy(t_hbm, t_v, isem)
    m_cp = pltpu.async_copy(m_hbm, m_v.at[pl.ds(0, N)], isem)
    pltpu.sync_copy(idx_hbm, idx_v.at[pl.ds(0, N)])
    idx_v[tail_sl] = jnp.where(lane < TAIL, idx_v[tail_sl], 0)
    for k in range(NG):
        sl = pl.ds(L * k, L)
        idx = idx_v[sl]
        row = ((idx >> 4) * DIV17_MAGIC) >> DIV17_SHIFT
        row_v[sl] = row
        col_v[sl] = idx - row * W
    t_cp.wait()
    m_cp.wait()
    m_v[tail_sl] = jnp.where(lane < TAIL, m_v[tail_sl], 0.0)

    def target(c, k):
        if k == NG - 1:
            return plsc.load_gather(t_v, [jnp.full((L,), c, jnp.int32), tail_col])
        return t_v[c, pl.ds(L * k, L)]

    acc = jnp.zeros((L,), jnp.float32)
    macc = jnp.zeros((L,), jnp.float32)
    cp0.wait()
    for k in range(NG):
        sl = pl.ds(L * k, L)
        m = m_v[sl]
        p0 = plsc.load_gather(ch_v, [row_v[sl], col_v[sl]])
        acc = acc + jnp.abs(p0 * m - target(0, k) * m)
        macc = macc + m

    cp1 = pltpu.async_copy(tab_hbm.at[1], ch_v, psem)
    cp1.wait()
    for k in range(NG):
        sl = pl.ds(L * k, L)
        m = m_v[sl]
        p1 = plsc.load_gather(ch_v, [row_v[sl], col_v[sl]])
        acc = acc + jnp.abs(p1 * m - target(1, k) * m)

    ltot = lane_total(acc)
    mtot = lane_total(macc)
    out_v[...] = ltot / (mtot + 0.0001)
    pltpu.sync_copy(out_v.at[pl.ds(0, 1)], out_hbm)


def kernel(output, centerFrame_index, center_index, mask):
    msk = mask.reshape(N)
    idx = center_index.reshape(N).astype(jnp.int32)
    out = _sc_l1_loss(output, centerFrame_index, msk, idx)
    return out.reshape(())


# speedup vs baseline: 1.3412x; 1.3412x over previous
import functools

import jax
import jax.numpy as jnp
from jax import lax
from jax.experimental import pallas as pl
from jax.experimental.pallas import tpu as pltpu
from jax.experimental.pallas import tpu_sc as plsc

H = 272
W = 272
C = 2
N = 500
HW = H * W
L = 16
NPAD = 512
TAIL = N - (NPAD - L)
NG = NPAD // L
CHUNK = 512
NCHUNK = NPAD // CHUNK

_mesh = plsc.VectorSubcoreMesh(
    core_axis_name="c", subcore_axis_name="s", num_cores=1, num_subcores=1
)


@functools.partial(
    pl.kernel,
    mesh=_mesh,
    out_type=jax.ShapeDtypeStruct((1,), jnp.float32),
    compiler_params=pltpu.CompilerParams(needs_layout_passes=False),
    scratch_types=[
        pltpu.VMEM((NPAD,), jnp.int32),
        pltpu.VMEM((2 * NPAD,), jnp.float32),
        pltpu.VMEM((C, N), jnp.float32),
        pltpu.VMEM((NPAD,), jnp.float32),
        pltpu.VMEM((L,), jnp.float32),
        pltpu.VMEM((L,), jnp.float32),
        pltpu.SemaphoreType.DMA,
        pltpu.SemaphoreType.DMA,
    ],
)
def _sc_l1_loss(tab_hbm, t_hbm, m_hbm, idx_hbm, out_hbm,
                idx_v, p_v, t_v, m_v, red_v, out_v, gsem, isem):

    def lane_total(x):
        for shift in (8, 4, 2, 1):
            red_v[...] = x
            perm = lax.iota(jnp.int32, L) ^ shift
            x = x + plsc.load_gather(red_v, [perm])
        return x

    lane = lax.iota(jnp.int32, L)
    tail_sl = pl.ds(NPAD - L, L)
    tail_col = jnp.where(lane < TAIL, (NPAD - L) + lane, 0)

    t_cp = pltpu.async_copy(t_hbm, t_v, isem)
    m_cp = pltpu.async_copy(m_hbm, m_v.at[pl.ds(0, N)], isem)
    pltpu.sync_copy(idx_hbm, idx_v.at[pl.ds(0, N)])
    idx_v[tail_sl] = jnp.where(lane < TAIL, idx_v[tail_sl], 0)
    tab1_hbm = tab_hbm.at[pl.ds(HW, HW)]
    copies = []
    for k in range(NCHUNK):
        isl = idx_v.at[pl.ds(CHUNK * k, CHUNK)]
        copies.append(
            pltpu.async_copy(tab_hbm.at[isl], p_v.at[pl.ds(CHUNK * k, CHUNK)], gsem)
        )
        copies.append(
            pltpu.async_copy(
                tab1_hbm.at[isl], p_v.at[pl.ds(NPAD + CHUNK * k, CHUNK)], gsem
            )
        )
    t_cp.wait()
    m_cp.wait()
    m_v[tail_sl] = jnp.where(lane < TAIL, m_v[tail_sl], 0.0)
    for cp in copies:
        cp.wait()

    def target(c, k):
        if k == NG - 1:
            return plsc.load_gather(t_v, [jnp.full((L,), c, jnp.int32), tail_col])
        return t_v[c, pl.ds(L * k, L)]

    acc = jnp.zeros((L,), jnp.float32)
    macc = jnp.zeros((L,), jnp.float32)
    for k in range(NG):
        sl = pl.ds(L * k, L)
        sl2 = pl.ds(NPAD + L * k, L)
        m = m_v[sl]
        acc = acc + jnp.abs(p_v[sl] * m - target(0, k) * m)
        acc = acc + jnp.abs(p_v[sl2] * m - target(1, k) * m)
        macc = macc + m
    ltot = lane_total(acc)
    mtot = lane_total(macc)
    out_v[...] = ltot / (mtot + 0.0001)
    pltpu.sync_copy(out_v.at[pl.ds(0, 1)], out_hbm)


def kernel(output, centerFrame_index, center_index, mask):
    tab = output.reshape(C * HW)
    msk = mask.reshape(N)
    idx = center_index.reshape(N).astype(jnp.int32)
    out = _sc_l1_loss(tab, centerFrame_index, msk, idx)
    return out.reshape(())
